# baseline (speedup 1.0000x reference)
import functools

import jax
import jax.numpy as jnp
from jax import lax
from jax.experimental import pallas as pl
from jax.experimental.pallas import tpu as pltpu
from jax.experimental.pallas import tpu_sc as plsc

N = 100000
D = 128
S = 128

BKR = 128
NB = 512
NTC = NB * BKR
BPG = 8
NBC = 32
SG = 8

NW = 32
CH = 1152
T = 288
NT = CH // T
NV = D // 16
G = 16
BPW = NB // NW
BIDS = BPW * BKR


def _sc_partials(h_flat, ids):
    mesh = plsc.VectorSubcoreMesh(core_axis_name="c", subcore_axis_name="s")

    @functools.partial(
        pl.kernel,
        mesh=mesh,
        out_type=jax.ShapeDtypeStruct((NW * S * D,), jnp.float32),
        scratch_types=[
            pltpu.VMEM((CH,), jnp.int32),
            pltpu.VMEM((BIDS,), jnp.int32),
            pltpu.VMEM((T * D,), jnp.float32),
            pltpu.VMEM((T * D,), jnp.float32),
            pltpu.VMEM((S * D,), jnp.float32),
            pltpu.SemaphoreType.DMA,
            pltpu.SemaphoreType.DMA,
        ],
    )
    def k(h_hbm, ids_hbm, out_hbm, ids_v, idsb_v, buf0, buf1, acc_v, sem0, sem1):
        wid = lax.axis_index("s") * 2 + lax.axis_index("c")
        base = NTC + ((wid * (N - NTC - CH)) // (NW - 1)) // 8 * 8
        base = pl.multiple_of(base, 8)
        bbase = wid * BIDS
        bufs = (buf0, buf1)
        sems = (sem0, sem1)

        def start_copy(t, b):
            pltpu.async_copy(
                h_hbm.at[pl.ds((base + t * T) * D, T * D)], bufs[b], sems[b]
            )

        def wait_copy(t, b):
            pltpu.make_async_copy(
                h_hbm.at[pl.ds((base + t * T) * D, T * D)], bufs[b], sems[b]
            ).wait()

        start_copy(0, 0)
        pltpu.sync_copy(ids_hbm.at[pl.ds(base, CH)], ids_v)
        pltpu.sync_copy(ids_hbm.at[pl.ds(bbase, BIDS)], idsb_v)

        neg = jnp.full((16,), -jnp.inf, dtype=jnp.float32)

        def init_blk(i, c):
            for u in range(8):
                acc_v[pl.ds(i * 128 + u * 16, 16)] = neg
            return c

        lax.fori_loop(0, S * D // 128, init_blk, 0)

        def tree_rmw(buf, row0, nrows, sid):
            for v in range(NV):
                vals = [
                    buf[pl.ds((row0 + r) * D + v * 16, 16)]
                    for r in range(nrows)
                ]
                while len(vals) > 1:
                    vals = [
                        jnp.maximum(vals[i], vals[i + 1])
                        for i in range(0, len(vals) - 1, 2)
                    ] + ([vals[-1]] if len(vals) % 2 else [])
                o = pl.ds(sid * D + v * 16, 16)
                acc_v[o] = jnp.maximum(acc_v[o], vals[0])

        def group16(buf, row0, idv):
            s0 = idv[0]
            uniform = s0 == idv[G - 1]

            @pl.when(uniform)
            def _():
                tree_rmw(buf, row0, G, s0)

            @pl.when(jnp.logical_not(uniform))
            def _():
                for r in range(G):
                    sid = idv[r]
                    for v in range(NV):
                        o = pl.ds(sid * D + v * 16, 16)
                        acc_v[o] = jnp.maximum(
                            acc_v[o], buf[pl.ds((row0 + r) * D + v * 16, 16)]
                        )

        def unit32(buf, idref, idoff, row0):
            idv0 = idref[pl.ds(idoff + row0, G)]
            idv1 = idref[pl.ds(idoff + row0 + G, G)]
            s0 = idv0[0]
            uniform = s0 == idv1[G - 1]

            @pl.when(uniform)
            def _():
                tree_rmw(buf, row0, 2 * G, s0)

            @pl.when(jnp.logical_not(uniform))
            def _():
                group16(buf, row0, idv0)
                group16(buf, row0 + G, idv1)

        def process(t, b):
            @pl.when(t + 1 < NT)
            def _():
                start_copy(t + 1, 1 - b)

            wait_copy(t, b)

            def unit(j, c):
                unit32(bufs[b], ids_v, t * T, j * 2 * G)
                return c

            lax.fori_loop(0, T // (2 * G), unit, 0)

        def pair(t, c):
            g = 2 * t
            process(g, 0)
            process(g + 1, 1)
            return c

        lax.fori_loop(0, NT // 2, pair, 0)

        def blk(kk, c):
            i0 = kk * BKR
            first = idsb_v[pl.ds(i0, 16)][0]
            last = idsb_v[pl.ds(i0 + BKR - 16, 16)][G - 1]

            @pl.when(first != last)
            def _():
                pltpu.sync_copy(
                    h_hbm.at[pl.ds((bbase + i0) * D, BKR * D)],
                    buf0.at[pl.ds(0, BKR * D)],
                )

                def bunit(j, c2):
                    unit32(buf0, idsb_v, i0, j * 2 * G)
                    return c2

                lax.fori_loop(0, BKR // (2 * G), bunit, 0)

            return c

        lax.fori_loop(0, BPW, blk, 0)

        pltpu.sync_copy(acc_v, out_hbm.at[pl.ds(wid * S * D, S * D)])

    return k(h_flat, ids)


def _tc_blockmax(h):
    def body(h_ref, o_ref):
        o_ref[...] = jnp.concatenate(
            [
                jnp.max(h_ref[pl.ds(k * BKR, BKR), :], axis=0, keepdims=True)
                for k in range(BPG)
            ],
            axis=0,
        )

    return pl.pallas_call(
        body,
        grid=(NB // BPG,),
        in_specs=[pl.BlockSpec((BPG * BKR, D), lambda i: (i, 0))],
        out_specs=pl.BlockSpec((BPG, D), lambda i: (i, 0)),
        out_shape=jax.ShapeDtypeStruct((NB, D), jnp.float32),
    )(h)


def _merge(partials_sc, bm, mask):
    def body(mask_ref, bm_ref, p_ref, o_ref):
        acc = jnp.max(p_ref[...], axis=0)
        for c in range(NB // NBC):
            mm = mask_ref[:, pl.ds(c * NBC, NBC)]
            bb = bm_ref[pl.ds(c * NBC, NBC), :]
            cand = jnp.where(mm[:, :, None] != 0, bb[None], -jnp.inf)
            acc = jnp.maximum(acc, jnp.max(cand, axis=1))
        o_ref[...] = acc

    return pl.pallas_call(
        body,
        grid=(S // SG,),
        in_specs=[
            pl.BlockSpec((SG, NB), lambda i: (i, 0)),
            pl.BlockSpec((NB, D), lambda i: (0, 0)),
            pl.BlockSpec((NW, SG, D), lambda i: (0, i, 0)),
        ],
        out_specs=pl.BlockSpec((SG, D), lambda i: (i, 0)),
        out_shape=jax.ShapeDtypeStruct((S, D), jnp.float32),
    )(mask, bm, partials_sc)


def kernel(h, segment_ids):
    idsb = segment_ids[:NTC].reshape(NB, BKR)
    first = idsb[:, 0]
    uni = first == idsb[:, -1]
    mask = (
        (first[None, :] == jnp.arange(S, dtype=jnp.int32)[:, None])
        & uni[None, :]
    ).astype(jnp.int32)

    partials_sc = _sc_partials(h.reshape(N * D), segment_ids)
    bm = _tc_blockmax(h)
    return _merge(partials_sc.reshape(NW, S, D), bm, mask)


# speedup vs baseline: 1.0361x; 1.0361x over previous
import functools

import jax
import jax.numpy as jnp
from jax import lax
from jax.experimental import pallas as pl
from jax.experimental.pallas import tpu as pltpu
from jax.experimental.pallas import tpu_sc as plsc

N = 100000
D = 128
S = 128

BKR = 128
NB = 448
NTC = NB * BKR
BPG = 8
SG = 8

NW = 32
CH = 1344
T = 224
NT = CH // T
NV = D // 16
G = 16
BPW = NB // NW
BIDS = BPW * BKR


def _sc_partials(h_flat, ids):
    mesh = plsc.VectorSubcoreMesh(core_axis_name="c", subcore_axis_name="s")

    @functools.partial(
        pl.kernel,
        mesh=mesh,
        out_type=jax.ShapeDtypeStruct((NW * S * D,), jnp.float32),
        scratch_types=[
            pltpu.VMEM((CH,), jnp.int32),
            pltpu.VMEM((BIDS,), jnp.int32),
            pltpu.VMEM((T * D,), jnp.float32),
            pltpu.VMEM((T * D,), jnp.float32),
            pltpu.VMEM((S * D,), jnp.float32),
            pltpu.SemaphoreType.DMA,
            pltpu.SemaphoreType.DMA,
        ],
    )
    def k(h_hbm, ids_hbm, out_hbm, ids_v, idsb_v, buf0, buf1, acc_v, sem0, sem1):
        wid = lax.axis_index("s") * 2 + lax.axis_index("c")
        base = NTC + ((wid * (N - NTC - CH)) // (NW - 1)) // 8 * 8
        base = pl.multiple_of(base, 8)
        bbase = wid * BIDS
        bufs = (buf0, buf1)
        sems = (sem0, sem1)

        def start_copy(t, b):
            pltpu.async_copy(
                h_hbm.at[pl.ds((base + t * T) * D, T * D)], bufs[b], sems[b]
            )

        def wait_copy(t, b):
            pltpu.make_async_copy(
                h_hbm.at[pl.ds((base + t * T) * D, T * D)], bufs[b], sems[b]
            ).wait()

        start_copy(0, 0)
        pltpu.sync_copy(ids_hbm.at[pl.ds(base, CH)], ids_v)
        pltpu.sync_copy(ids_hbm.at[pl.ds(bbase, BIDS)], idsb_v)

        neg = jnp.full((16,), -jnp.inf, dtype=jnp.float32)

        def init_blk(i, c):
            for u in range(8):
                acc_v[pl.ds(i * 128 + u * 16, 16)] = neg
            return c

        lax.fori_loop(0, S * D // 128, init_blk, 0)

        def tree_rmw(buf, row0, nrows, sid):
            for v in range(NV):
                vals = [
                    buf[pl.ds((row0 + r) * D + v * 16, 16)]
                    for r in range(nrows)
                ]
                while len(vals) > 1:
                    vals = [
                        jnp.maximum(vals[i], vals[i + 1])
                        for i in range(0, len(vals) - 1, 2)
                    ] + ([vals[-1]] if len(vals) % 2 else [])
                o = pl.ds(sid * D + v * 16, 16)
                acc_v[o] = jnp.maximum(acc_v[o], vals[0])

        def group16(buf, row0, idv):
            s0 = idv[0]
            uniform = s0 == idv[G - 1]

            @pl.when(uniform)
            def _():
                tree_rmw(buf, row0, G, s0)

            @pl.when(jnp.logical_not(uniform))
            def _():
                for r in range(G):
                    sid = idv[r]
                    for v in range(NV):
                        o = pl.ds(sid * D + v * 16, 16)
                        acc_v[o] = jnp.maximum(
                            acc_v[o], buf[pl.ds((row0 + r) * D + v * 16, 16)]
                        )

        def unit32(buf, idref, idoff, row0):
            idv0 = idref[pl.ds(idoff + row0, G)]
            idv1 = idref[pl.ds(idoff + row0 + G, G)]
            s0 = idv0[0]
            uniform = s0 == idv1[G - 1]

            @pl.when(uniform)
            def _():
                tree_rmw(buf, row0, 2 * G, s0)

            @pl.when(jnp.logical_not(uniform))
            def _():
                group16(buf, row0, idv0)
                group16(buf, row0 + G, idv1)

        def process(t, b):
            @pl.when(t + 1 < NT)
            def _():
                start_copy(t + 1, 1 - b)

            wait_copy(t, b)

            def unit(j, c):
                unit32(bufs[b], ids_v, t * T, j * 2 * G)
                return c

            lax.fori_loop(0, T // (2 * G), unit, 0)

        def pair(t, c):
            g = 2 * t
            process(g, 0)
            process(g + 1, 1)
            return c

        lax.fori_loop(0, NT // 2, pair, 0)

        def blk(kk, c):
            i0 = kk * BKR
            first = idsb_v[pl.ds(i0, 16)][0]
            last = idsb_v[pl.ds(i0 + BKR - 16, 16)][G - 1]

            @pl.when(first != last)
            def _():
                pltpu.sync_copy(
                    h_hbm.at[pl.ds((bbase + i0) * D, BKR * D)],
                    buf0.at[pl.ds(0, BKR * D)],
                )

                def bunit(j, c2):
                    unit32(buf0, idsb_v, i0, j * 2 * G)
                    return c2

                lax.fori_loop(0, BKR // (2 * G), bunit, 0)

            return c

        lax.fori_loop(0, BPW, blk, 0)

        pltpu.sync_copy(acc_v, out_hbm.at[pl.ds(wid * S * D, S * D)])

    return k(h_flat, ids)


def _tc_blockmax(h):
    def body(h_ref, o_ref):
        o_ref[...] = jnp.concatenate(
            [
                jnp.max(h_ref[pl.ds(k * BKR, BKR), :], axis=0, keepdims=True)
                for k in range(BPG)
            ],
            axis=0,
        )

    return pl.pallas_call(
        body,
        grid=(NB // BPG,),
        in_specs=[pl.BlockSpec((BPG * BKR, D), lambda i: (i, 0))],
        out_specs=pl.BlockSpec((BPG, D), lambda i: (i, 0)),
        out_shape=jax.ShapeDtypeStruct((NB, D), jnp.float32),
    )(h)


def _merge(partials_sc, bm, mask_t):
    def body(mask_ref, bm_ref, p_ref, o_ref):
        acc = jnp.max(p_ref[...], axis=0)
        outs = []
        for s in range(SG):
            col = mask_ref[0, :, pl.ds(s, 1)]
            cand = jnp.where(col != 0, bm_ref[...], -jnp.inf)
            outs.append(jnp.max(cand, axis=0, keepdims=True))
        o_ref[...] = jnp.maximum(acc, jnp.concatenate(outs, axis=0))

    return pl.pallas_call(
        body,
        grid=(S // SG,),
        in_specs=[
            pl.BlockSpec((1, NB, SG), lambda i: (i, 0, 0)),
            pl.BlockSpec((NB, D), lambda i: (0, 0)),
            pl.BlockSpec((NW, SG, D), lambda i: (0, i, 0)),
        ],
        out_specs=pl.BlockSpec((SG, D), lambda i: (i, 0)),
        out_shape=jax.ShapeDtypeStruct((S, D), jnp.float32),
    )(mask_t, bm, partials_sc)


def kernel(h, segment_ids):
    idsb = segment_ids[:NTC].reshape(NB, BKR)
    first = idsb[:, 0]
    uni = first == idsb[:, -1]
    mask_t = (
        (first[:, None] == jnp.arange(S, dtype=jnp.int32)[None, :])
        & uni[:, None]
    ).astype(jnp.int32)
    mask3 = mask_t.reshape(NB, S // SG, SG).transpose(1, 0, 2)

    partials_sc = _sc_partials(h.reshape(N * D), segment_ids)
    bm = _tc_blockmax(h)
    return _merge(partials_sc.reshape(NW, S, D), bm, mask3)


# speedup vs baseline: 1.1702x; 1.1294x over previous
import functools

import jax
import jax.numpy as jnp
from jax import lax
from jax.experimental import pallas as pl
from jax.experimental.pallas import tpu as pltpu
from jax.experimental.pallas import tpu_sc as plsc

N = 100000
D = 128
S = 128
NW = 32
CH = 3136
T = 224
NT = CH // T
NV = D // 16
G = 16
NG = T // G


def _sc_partials(h_flat, ids):
    mesh = plsc.VectorSubcoreMesh(core_axis_name="c", subcore_axis_name="s")

    @functools.partial(
        pl.kernel,
        mesh=mesh,
        out_type=jax.ShapeDtypeStruct((NW * S * D,), jnp.float32),
        scratch_types=[
            pltpu.VMEM((CH,), jnp.int32),
            pltpu.VMEM((T * D,), jnp.float32),
            pltpu.VMEM((T * D,), jnp.float32),
            pltpu.VMEM((S * D,), jnp.float32),
            pltpu.SemaphoreType.DMA,
            pltpu.SemaphoreType.DMA,
        ],
    )
    def k(h_hbm, ids_hbm, out_hbm, ids_v, buf0, buf1, acc_v, sem0, sem1):
        wid = lax.axis_index("s") * 2 + lax.axis_index("c")
        base = ((wid * (N - CH)) // (NW - 1)) // 8 * 8
        base = pl.multiple_of(base, 8)
        bufs = (buf0, buf1)
        sems = (sem0, sem1)

        def start_copy(t, b):
            pltpu.async_copy(
                h_hbm.at[pl.ds((base + t * T) * D, T * D)], bufs[b], sems[b]
            )

        def wait_copy(t, b):
            pltpu.make_async_copy(
                h_hbm.at[pl.ds((base + t * T) * D, T * D)], bufs[b], sems[b]
            ).wait()

        start_copy(0, 0)
        pltpu.sync_copy(ids_hbm.at[pl.ds(base, CH)], ids_v)

        neg = jnp.full((16,), -jnp.inf, dtype=jnp.float32)

        def init_blk(i, c):
            for u in range(8):
                acc_v[pl.ds(i * 128 + u * 16, 16)] = neg
            return c

        lax.fori_loop(0, S * D // 128, init_blk, 0)

        def process(t, b):
            @pl.when(t + 1 < NT)
            def _():
                start_copy(t + 1, 1 - b)

            wait_copy(t, b)
            buf = bufs[b]

            def tree_rmw(row0, nrows, sid):
                for v in range(NV):
                    vals = [
                        buf[pl.ds((row0 + r) * D + v * 16, 16)]
                        for r in range(nrows)
                    ]
                    while len(vals) > 1:
                        vals = [
                            jnp.maximum(vals[i], vals[i + 1])
                            for i in range(0, len(vals) - 1, 2)
                        ] + ([vals[-1]] if len(vals) % 2 else [])
                    o = pl.ds(sid * D + v * 16, 16)
                    acc_v[o] = jnp.maximum(acc_v[o], vals[0])

            def group16(row0, idv):
                s0 = idv[0]
                uniform = s0 == idv[G - 1]

                @pl.when(uniform)
                def _():
                    tree_rmw(row0, G, s0)

                @pl.when(jnp.logical_not(uniform))
                def _():
                    for r in range(G):
                        sid = idv[r]
                        for v in range(NV):
                            o = pl.ds(sid * D + v * 16, 16)
                            acc_v[o] = jnp.maximum(
                                acc_v[o], buf[pl.ds((row0 + r) * D + v * 16, 16)]
                            )

            def unit(j, c):
                row0 = j * 2 * G
                idv0 = ids_v[pl.ds(t * T + row0, G)]
                idv1 = ids_v[pl.ds(t * T + row0 + G, G)]
                s0 = idv0[0]
                uniform = s0 == idv1[G - 1]

                @pl.when(uniform)
                def _():
                    tree_rmw(row0, 2 * G, s0)

                @pl.when(jnp.logical_not(uniform))
                def _():
                    group16(row0, idv0)
                    group16(row0 + G, idv1)

                return c

            lax.fori_loop(0, NG // 2, unit, 0)

        def pair(t, c):
            g = 2 * t
            process(g, 0)
            process(g + 1, 1)
            return c

        lax.fori_loop(0, NT // 2, pair, 0)

        pltpu.sync_copy(acc_v, out_hbm.at[pl.ds(wid * S * D, S * D)])

    return k(h_flat, ids)


def _merge(partials):
    def body(p_ref, o_ref):
        o_ref[...] = jnp.max(p_ref[...], axis=0)

    return pl.pallas_call(
        body,
        out_shape=jax.ShapeDtypeStruct((S, D), jnp.float32),
    )(partials)


def kernel(h, segment_ids):
    partials = _sc_partials(h.reshape(N * D), segment_ids)
    return _merge(partials.reshape(NW, S, D))
